# Initial kernel scaffold; baseline (speedup 1.0000x reference)
import jax
import jax.numpy as jnp
from jax.experimental import pallas as pl


def kernel(x, ei0, ei1, ei2, ew0, ew1, ew2, pe, W_in, b_in, d, hop_bias, Wf0, bf0, Wf1, bf1, Wf2, bf2):
    raise NotImplementedError("write your pallas kernel here")


# speedup vs baseline: 4.7366x; 4.7366x over previous
import functools

import jax
import jax.numpy as jnp
from jax import lax
from jax.experimental import pallas as pl
from jax.experimental.pallas import tpu as pltpu
from jax.experimental.pallas import tpu_sc as plsc

N = 10000
E = 320000
C = 128
H = 32
W3 = 3 * H
K = 80
NT = 16
EPT = E // NT
NCH = EPT // K
RPT = N // NT


def _xt_body(x_ref, w_ref, b_ref, o_ref):
    o_ref[...] = (
        jnp.dot(x_ref[...], w_ref[...], preferred_element_type=jnp.float32)
        + b_ref[...]
    )


@jax.jit
def _xt_call(x, wt, b):
    return pl.pallas_call(
        _xt_body,
        grid=(10,),
        in_specs=[
            pl.BlockSpec((N // 10, C), lambda i: (i, 0)),
            pl.BlockSpec((C, C), lambda i: (0, 0)),
            pl.BlockSpec((1, C), lambda i: (0, 0)),
        ],
        out_specs=pl.BlockSpec((N // 10, C), lambda i: (i, 0)),
        out_shape=jax.ShapeDtypeStruct((N, C), jnp.float32),
    )(x, wt, b)


_MESH = plsc.VectorSubcoreMesh(core_axis_name="c", subcore_axis_name="s")


def _make_hop(q, with_deg):
    out_type = [jax.ShapeDtypeStruct((2 * N, W3), jnp.float32)]
    scratch = [
        pltpu.VMEM_SHARED((N, W3), jnp.float32),
        pltpu.VMEM((N,), jnp.float32),
        pltpu.VMEM((K,), jnp.int32),
        pltpu.VMEM((K,), jnp.int32),
        pltpu.VMEM((K,), jnp.int32),
        pltpu.VMEM((K,), jnp.float32),
        pltpu.VMEM((K,), jnp.float32),
        pltpu.VMEM((K,), jnp.float32),
        pltpu.VMEM((K, H), jnp.float32),
        pltpu.VMEM((K, W3), jnp.float32),
        pltpu.VMEM((K, 16), jnp.float32),
        pltpu.SemaphoreType.DMA,
    ]
    if with_deg:
        out_type.append(jax.ShapeDtypeStruct((N, 16), jnp.float32))
        scratch.insert(1, pltpu.VMEM_SHARED((N, 16), jnp.float32))

    def body(*refs):
        it = iter(refs)
        xt_hbm, ei_hbm, ew_hbm, pe_hbm, acc_hbm = (next(it) for _ in range(5))
        deg_hbm = next(it) if with_deg else None
        acc_s = next(it)
        deg_s = next(it) if with_deg else None
        (pe_v, row_v, col_v, idx_v, ew_v, wu_v, wd_v,
         xr_v, msg_v, dmsg_v, sem) = it

        core = lax.axis_index("c")
        tid = lax.axis_index("s")
        zero16 = jnp.zeros((16,), jnp.float32)
        l16 = lax.iota(jnp.int32, 16)

        @pl.loop(0, K)
        def _zb(k):
            for c in range(W3 // 16):
                msg_v[k, pl.ds(16 * c, 16)] = zero16
            dmsg_v[k, :] = zero16

        r0 = tid * RPT
        nfull = RPT // K
        rem = RPT % K
        for j in range(nfull):
            pltpu.sync_copy(msg_v, acc_s.at[pl.ds(r0 + K * j, K)])
            if with_deg:
                pltpu.sync_copy(dmsg_v, deg_s.at[pl.ds(r0 + K * j, K)])
        if rem:
            pltpu.sync_copy(msg_v.at[pl.ds(0, rem)],
                            acc_s.at[pl.ds(r0 + RPT - rem, rem)])
            if with_deg:
                pltpu.sync_copy(dmsg_v.at[pl.ds(0, rem)],
                                deg_s.at[pl.ds(r0 + RPT - rem, rem)])

        pltpu.sync_copy(pe_hbm, pe_v)
        plsc.subcore_barrier()

        ebase = tid * EPT
        qoff = 2 * q + core

        @pl.loop(0, NCH)
        def _chunk(i):
            b = ebase + i * K
            pltpu.sync_copy(ei_hbm.at[0, pl.ds(b, K)], row_v)
            pltpu.sync_copy(ei_hbm.at[1, pl.ds(b, K)], col_v)
            pltpu.sync_copy(ew_hbm.at[pl.ds(b, K)], ew_v)

            for j in range(K // 16):
                cv = col_v[pl.ds(16 * j, 16)]
                rv = row_v[pl.ds(16 * j, 16)]
                idx_v[pl.ds(16 * j, 16)] = cv * 4 + qoff
                diff = (plsc.load_gather(pe_v, [cv])
                        - plsc.load_gather(pe_v, [rv]))
                wu_v[pl.ds(16 * j, 16)] = jnp.maximum(diff, 0.0)
                wd_v[pl.ds(16 * j, 16)] = jnp.maximum(-diff, 0.0)

            pltpu.async_copy(xt_hbm.at[idx_v], xr_v, sem).wait()

            @pl.loop(0, K, unroll=8)
            def _edge(k):
                ks = jnp.full((16,), k, jnp.int32)
                ev = plsc.load_gather(ew_v, [ks])
                uv = plsc.load_gather(wu_v, [ks])
                dv = plsc.load_gather(wd_v, [ks])
                for c in range(H // 16):
                    xv = xr_v[k, pl.ds(16 * c, 16)]
                    msg_v[k, pl.ds(16 * c, 16)] = xv * ev
                    msg_v[k, pl.ds(H + 16 * c, 16)] = xv * uv
                    msg_v[k, pl.ds(2 * H + 16 * c, 16)] = xv * dv
                if with_deg:
                    dmsg_v[k, :] = jnp.where(l16 == 0, ev, zero16)

            pltpu.sync_copy(msg_v, acc_s.at[row_v], add=True)
            if with_deg:
                @pl.when(core == 0)
                def _():
                    pltpu.sync_copy(dmsg_v, deg_s.at[row_v], add=True)

        plsc.subcore_barrier()

        def _wb(src, dst, off):
            for j in range(nfull):
                pltpu.sync_copy(src.at[pl.ds(r0 + K * j, K)],
                                dst.at[pl.ds(off + r0 + K * j, K)])
            if rem:
                pltpu.sync_copy(src.at[pl.ds(r0 + RPT - rem, rem)],
                                dst.at[pl.ds(off + r0 + RPT - rem, rem)])

        _wb(acc_s, acc_hbm, core * N)

        if with_deg:
            @pl.when(core == 0)
            def _():
                _wb(deg_s, deg_hbm, 0)

    return pl.kernel(
        body,
        out_type=tuple(out_type),
        mesh=_MESH,
        compiler_params=pltpu.CompilerParams(use_tc_tiling_on_sc=False,
                                             needs_layout_passes=False),
        scratch_types=scratch,
    )


_hop_q0 = _make_hop(0, True)
_hop_q1 = _make_hop(1, False)


def _fuse_body(*refs):
    (xt_ref, *rest) = refs
    a = rest[:12]
    dg = rest[12:15]
    wx_ref, sp_ref, bf_ref, dl_ref, hb_ref, o_ref = rest[15:]

    x = xt_ref[...]
    dl = dl_ref[...]
    m = jnp.max(dl, axis=0, keepdims=True)
    e = jnp.exp(dl - m)
    dw = e / jnp.sum(e, axis=0, keepdims=True)

    final = jnp.zeros_like(x)
    for p in range(3):
        deg = dg[p][...][:, 0:1]
        dinv = jnp.where(deg == 0.0, 0.0, 1.0 / deg)
        z = jnp.dot(x, wx_ref[p], preferred_element_type=jnp.float32)
        for u in range(4):
            z = z + jnp.dot(a[4 * p + u][...] * dinv, sp_ref[p, u],
                            preferred_element_type=jnp.float32)
        z = z + bf_ref[p]
        h = jnp.where(z >= 0.0, z, 0.01 * z)
        final = final + h * dw[p] + hb_ref[p]
    o_ref[...] = final


@jax.jit
def _fuse_call(xt, a_in, degs, wx, sp, bf, dl, hb):
    B = N // 10
    row_spec = lambda w: pl.BlockSpec((B, w), lambda i: (i, 0))
    full = lambda s: pl.BlockSpec(s, lambda i: tuple(0 for _ in s))
    return pl.pallas_call(
        _fuse_body,
        grid=(10,),
        in_specs=(
            [row_spec(C)]
            + [row_spec(W3)] * 12
            + [row_spec(16)] * 3
            + [full((3, C, C)), full((3, 4, W3, C)), full((3, 1, C)),
               full((3, C)), full((3, 1, C))]
        ),
        out_specs=row_spec(C),
        out_shape=jax.ShapeDtypeStruct((N, C), jnp.float32),
    )(xt, *a_in, *degs, wx, sp, bf, dl, hb)


@jax.jit
def kernel(x, ei0, ei1, ei2, ew0, ew1, ew2, pe,
           W_in, b_in, d, hop_bias, Wf0, bf0, Wf1, bf1, Wf2, bf2):
    xt = _xt_call(x, W_in.T, b_in.reshape(1, C))
    xt4 = xt.reshape(4 * N, H)
    pe0 = pe[:, 0]

    a_in, degs = [], []
    for ei, ew in ((ei0, ew0), (ei1, ew1), (ei2, ew2)):
        acc0, deg = _hop_q0(xt4, ei, ew, pe0)
        (acc1,) = _hop_q1(xt4, ei, ew, pe0)
        a_in += [acc0[:N], acc0[N:], acc1[:N], acc1[N:]]
        degs.append(deg)

    wx, sp, bf = [], [], []
    for Wf, bfp in ((Wf0, bf0), (Wf1, bf1), (Wf2, bf2)):
        wx.append(Wf[:, :C].T)
        Wa = Wf[:, C:2 * C].T
        Wu = Wf[:, 2 * C::2].T
        Wd = Wf[:, 2 * C + 1::2].T
        sp.append(jnp.stack([
            jnp.concatenate([Wa[H * u:H * (u + 1)],
                             Wu[H * u:H * (u + 1)],
                             Wd[H * u:H * (u + 1)]], axis=0)
            for u in range(4)
        ]))
        bf.append(bfp.reshape(1, C))
    wx = jnp.stack(wx)
    sp = jnp.stack(sp)
    bf = jnp.stack(bf)
    hb = hop_bias.reshape(3, 1, C)

    return _fuse_call(xt, a_in, degs, wx, sp, bf, d, hb)


# speedup vs baseline: 10.2943x; 2.1734x over previous
import functools

import jax
import jax.numpy as jnp
from jax import lax
from jax.experimental import pallas as pl
from jax.experimental.pallas import tpu as pltpu
from jax.experimental.pallas import tpu_sc as plsc

N = 10000
E = 320000
C = 128
H = 32
W3 = 3 * H
K = 80
NT = 16
EPT = E // NT
NCH = EPT // K
NSL = 2
T = NCH // NSL
RPT = N // NT


def _xt_body(x_ref, w_ref, b_ref, o_ref):
    o_ref[...] = (
        jnp.dot(x_ref[...], w_ref[...], preferred_element_type=jnp.float32)
        + b_ref[...]
    )


@jax.jit
def _xt_call(x, wt, b):
    return pl.pallas_call(
        _xt_body,
        grid=(10,),
        in_specs=[
            pl.BlockSpec((N // 10, C), lambda i: (i, 0)),
            pl.BlockSpec((C, C), lambda i: (0, 0)),
            pl.BlockSpec((1, C), lambda i: (0, 0)),
        ],
        out_specs=pl.BlockSpec((N // 10, C), lambda i: (i, 0)),
        out_shape=jax.ShapeDtypeStruct((N, C), jnp.float32),
    )(x, wt, b)


_MESH = plsc.VectorSubcoreMesh(core_axis_name="c", subcore_axis_name="s")


def _make_hop(q, with_deg):
    out_type = [jax.ShapeDtypeStruct((2 * N, W3), jnp.float32)]
    scratch = [
        pltpu.VMEM_SHARED((N, W3), jnp.float32),
        pltpu.VMEM((N,), jnp.float32),
        pltpu.VMEM((NSL, 3, K), jnp.int32),
        pltpu.VMEM((NSL, K), jnp.int32),
        pltpu.VMEM((NSL, K), jnp.int32),
        pltpu.VMEM((NSL, K), jnp.float32),
        pltpu.VMEM((NSL, K), jnp.float32),
        pltpu.VMEM((NSL, K), jnp.float32),
        pltpu.VMEM((NSL, K, H), jnp.float32),
        pltpu.VMEM((NSL, K, W3), jnp.float32),
        pltpu.VMEM((NSL, K, 16), jnp.float32),
    ] + [pltpu.SemaphoreType.DMA] * (3 * NSL) \
      + ([pltpu.SemaphoreType.DMA] * NSL if with_deg else [])
    if with_deg:
        out_type.append(jax.ShapeDtypeStruct((N, 16), jnp.float32))
        scratch.insert(1, pltpu.VMEM_SHARED((N, 16), jnp.float32))

    def body(*refs):
        it = iter(refs)
        xt_hbm, pk_hbm, pe_hbm, acc_hbm = (next(it) for _ in range(4))
        deg_hbm = next(it) if with_deg else None
        acc_s = next(it)
        deg_s = next(it) if with_deg else None
        (pe_v, pk_v, idx_v, ridx_v, ewc_v, wu_v, wd_v,
         xr_v, msg_v, dmsg_v) = (next(it) for _ in range(10))
        rest = list(it)
        sin = rest[0:NSL]
        sg = rest[NSL:2 * NSL]
        ss = rest[2 * NSL:3 * NSL]
        sd = rest[3 * NSL:] if with_deg else None

        core = lax.axis_index("c")
        tid = lax.axis_index("s")
        zero16 = jnp.zeros((16,), jnp.float32)
        l16 = lax.iota(jnp.int32, 16)
        ebase = tid * EPT
        qoff = 2 * q + core

        def loads_copy(ci, s):
            return pltpu.make_async_copy(
                pk_hbm.at[:, pl.ds(ebase + ci * K, K)], pk_v.at[s], sin[s])

        def gather_copy(s):
            return pltpu.make_async_copy(
                xt_hbm.at[idx_v.at[s]], xr_v.at[s], sg[s])

        def scat_copy(s):
            return pltpu.make_async_copy(
                msg_v.at[s], acc_s.at[ridx_v.at[s]], ss[s])

        def dscat_copy(s):
            return pltpu.make_async_copy(
                dmsg_v.at[s], deg_s.at[ridx_v.at[s]], sd[s])

        def prep(s):
            for j in range(K // 16):
                sl = pl.ds(16 * j, 16)
                rv = pk_v[s, 0, sl]
                cv = pk_v[s, 1, sl]
                ridx_v[s, sl] = rv
                idx_v[s, sl] = cv * 4 + qoff
                ewc_v[s, sl] = plsc.bitcast(pk_v[s, 2, sl], jnp.float32)
                diff = (plsc.load_gather(pe_v, [cv])
                        - plsc.load_gather(pe_v, [rv]))
                wu_v[s, sl] = jnp.maximum(diff, 0.0)
                wd_v[s, sl] = jnp.maximum(-diff, 0.0)

        def edge_build(s):
            @pl.loop(0, K, unroll=8)
            def _edge(k):
                ks = jnp.full((16,), k, jnp.int32)
                ev = plsc.load_gather(ewc_v.at[s], [ks])
                uv = plsc.load_gather(wu_v.at[s], [ks])
                dv = plsc.load_gather(wd_v.at[s], [ks])
                for c in range(H // 16):
                    xv = xr_v[s, k, pl.ds(16 * c, 16)]
                    msg_v[s, k, pl.ds(16 * c, 16)] = xv * ev
                    msg_v[s, k, pl.ds(H + 16 * c, 16)] = xv * uv
                    msg_v[s, k, pl.ds(2 * H + 16 * c, 16)] = xv * dv
                if with_deg:
                    dmsg_v[s, k, :] = jnp.where(l16 == 0, ev, zero16)

        def issue_scatter(s):
            pltpu.sync_copy(msg_v.at[s], acc_s.at[ridx_v.at[s]], add=True)
            if with_deg:
                @pl.when(core == 0)
                def _():
                    pltpu.sync_copy(dmsg_v.at[s], deg_s.at[ridx_v.at[s]],
                                    add=True)

        def wait_scatter(s):
            pass

        for s in range(NSL):
            loads_copy(s, s).start()

        @pl.loop(0, K)
        def _zb(k):
            for c in range(W3 // 16):
                msg_v[0, k, pl.ds(16 * c, 16)] = zero16
            dmsg_v[0, k, :] = zero16

        r0 = tid * RPT
        nfull = RPT // K
        rem = RPT % K
        for j in range(nfull):
            pltpu.sync_copy(msg_v.at[0], acc_s.at[pl.ds(r0 + K * j, K)])
            if with_deg:
                pltpu.sync_copy(dmsg_v.at[0], deg_s.at[pl.ds(r0 + K * j, K)])
        if rem:
            pltpu.sync_copy(msg_v.at[0].at[pl.ds(0, rem)],
                            acc_s.at[pl.ds(r0 + RPT - rem, rem)])
            if with_deg:
                pltpu.sync_copy(dmsg_v.at[0].at[pl.ds(0, rem)],
                                deg_s.at[pl.ds(r0 + RPT - rem, rem)])

        pltpu.sync_copy(pe_hbm, pe_v)
        plsc.subcore_barrier()

        loads_copy(0, 0).wait()
        prep(0)
        gather_copy(0).start()

        @pl.loop(0, T)
        def _step(t):
            for b in range(NSL):
                i = NSL * t + b
                s1 = (b + 1) % NSL

                def prep_next():
                    loads_copy(i + 1, s1).wait()
                    prep(s1)
                    gather_copy(s1).start()

                if b < NSL - 1:
                    prep_next()
                else:
                    pl.when(t < T - 1)(prep_next)

                pl.when(t < T - 1)(lambda: loads_copy(i + 2, b).start())

                gather_copy(b).wait()
                edge_build(b)
                issue_scatter(b)

        plsc.subcore_barrier()

        def _wb(src, dst, off):
            for j in range(nfull):
                pltpu.sync_copy(src.at[pl.ds(r0 + K * j, K)],
                                dst.at[pl.ds(off + r0 + K * j, K)])
            if rem:
                pltpu.sync_copy(src.at[pl.ds(r0 + RPT - rem, rem)],
                                dst.at[pl.ds(off + r0 + RPT - rem, rem)])

        _wb(acc_s, acc_hbm, core * N)

        if with_deg:
            @pl.when(core == 0)
            def _():
                _wb(deg_s, deg_hbm, 0)

    return pl.kernel(
        body,
        out_type=tuple(out_type),
        mesh=_MESH,
        compiler_params=pltpu.CompilerParams(use_tc_tiling_on_sc=False,
                                             needs_layout_passes=False),
        scratch_types=scratch,
    )


_hop_q0 = _make_hop(0, True)
_hop_q1 = _make_hop(1, False)


def _fuse_body(*refs):
    (xt_ref, *rest) = refs
    a = rest[:12]
    dg = rest[12:15]
    wx_ref, sp_ref, bf_ref, dl_ref, hb_ref, o_ref = rest[15:]

    x = xt_ref[...]
    dl = dl_ref[...]
    m = jnp.max(dl, axis=0, keepdims=True)
    e = jnp.exp(dl - m)
    dw = e / jnp.sum(e, axis=0, keepdims=True)

    final = jnp.zeros_like(x)
    for p in range(3):
        deg = dg[p][...][:, 0:1]
        dinv = jnp.where(deg == 0.0, 0.0, 1.0 / deg)
        z = jnp.dot(x, wx_ref[p], preferred_element_type=jnp.float32)
        for u in range(4):
            z = z + jnp.dot(a[4 * p + u][...] * dinv, sp_ref[p, u],
                            preferred_element_type=jnp.float32)
        z = z + bf_ref[p]
        h = jnp.where(z >= 0.0, z, 0.01 * z)
        final = final + h * dw[p] + hb_ref[p]
    o_ref[...] = final


@jax.jit
def _fuse_call(xt, a_in, degs, wx, sp, bf, dl, hb):
    B = N // 10
    row_spec = lambda w: pl.BlockSpec((B, w), lambda i: (i, 0))
    full = lambda s: pl.BlockSpec(s, lambda i: tuple(0 for _ in s))
    return pl.pallas_call(
        _fuse_body,
        grid=(10,),
        in_specs=(
            [row_spec(C)]
            + [row_spec(W3)] * 12
            + [row_spec(16)] * 3
            + [full((3, C, C)), full((3, 4, W3, C)), full((3, 1, C)),
               full((3, C)), full((3, 1, C))]
        ),
        out_specs=row_spec(C),
        out_shape=jax.ShapeDtypeStruct((N, C), jnp.float32),
    )(xt, *a_in, *degs, wx, sp, bf, dl, hb)


@jax.jit
def kernel(x, ei0, ei1, ei2, ew0, ew1, ew2, pe,
           W_in, b_in, d, hop_bias, Wf0, bf0, Wf1, bf1, Wf2, bf2):
    xt = _xt_call(x, W_in.T, b_in.reshape(1, C))
    xt4 = xt.reshape(4 * N, H)
    pe0 = pe[:, 0]

    a_in, degs = [], []
    for ei, ew in ((ei0, ew0), (ei1, ew1), (ei2, ew2)):
        pk = jnp.concatenate(
            [ei, lax.bitcast_convert_type(ew, jnp.int32)[None]], axis=0)
        acc0, deg = _hop_q0(xt4, pk, pe0)
        (acc1,) = _hop_q1(xt4, pk, pe0)
        a_in += [acc0[:N], acc0[N:], acc1[:N], acc1[N:]]
        degs.append(deg)

    wx, sp, bf = [], [], []
    for Wf, bfp in ((Wf0, bf0), (Wf1, bf1), (Wf2, bf2)):
        wx.append(Wf[:, :C].T)
        Wa = Wf[:, C:2 * C].T
        Wu = Wf[:, 2 * C::2].T
        Wd = Wf[:, 2 * C + 1::2].T
        sp.append(jnp.stack([
            jnp.concatenate([Wa[H * u:H * (u + 1)],
                             Wu[H * u:H * (u + 1)],
                             Wd[H * u:H * (u + 1)]], axis=0)
            for u in range(4)
        ]))
        bf.append(bfp.reshape(1, C))
    wx = jnp.stack(wx)
    sp = jnp.stack(sp)
    bf = jnp.stack(bf)
    hb = hop_bias.reshape(3, 1, C)

    return _fuse_call(xt, a_in, degs, wx, sp, bf, d, hb)


# speedup vs baseline: 13.7129x; 1.3321x over previous
import functools

import jax
import jax.numpy as jnp
from jax import lax
from jax.experimental import pallas as pl
from jax.experimental.pallas import tpu as pltpu
from jax.experimental.pallas import tpu_sc as plsc

N = 10000
E = 320000
C = 128
H = 32
W3 = 3 * H
K = 80
NT = 16
EPT = E // NT
NCH = EPT // K
NSL = 2
T = NCH // NSL
RPT = N // NT


def _xt_body(x_ref, w_ref, b_ref, o_ref):
    o_ref[...] = (
        jnp.dot(x_ref[...], w_ref[...], preferred_element_type=jnp.float32)
        + b_ref[...]
    )


@jax.jit
def _xt_call(x, wt, b):
    return pl.pallas_call(
        _xt_body,
        grid=(10,),
        in_specs=[
            pl.BlockSpec((N // 10, C), lambda i: (i, 0)),
            pl.BlockSpec((C, C), lambda i: (0, 0)),
            pl.BlockSpec((1, C), lambda i: (0, 0)),
        ],
        out_specs=pl.BlockSpec((N // 10, C), lambda i: (i, 0)),
        out_shape=jax.ShapeDtypeStruct((N, C), jnp.float32),
    )(x, wt, b)


_MESH = plsc.VectorSubcoreMesh(core_axis_name="c", subcore_axis_name="s")


def _make_hop(q, with_deg):
    W = W3 + 16 if with_deg else W3
    out_type = jax.ShapeDtypeStruct((2 * N, W), jnp.float32)
    scratch = [
        pltpu.VMEM_SHARED((N, W), jnp.float32),
        pltpu.VMEM((N,), jnp.float32),
        pltpu.VMEM((NSL, 3, K), jnp.int32),
        pltpu.VMEM((NSL, K), jnp.int32),
        pltpu.VMEM((NSL, K), jnp.int32),
        pltpu.VMEM((NSL, K), jnp.float32),
        pltpu.VMEM((NSL, K), jnp.float32),
        pltpu.VMEM((NSL, K), jnp.float32),
        pltpu.VMEM((NSL, K, H), jnp.float32),
        pltpu.VMEM((NSL, K, W), jnp.float32),
    ] + [pltpu.SemaphoreType.DMA] * (3 * NSL)

    def body(*refs):
        it = iter(refs)
        xt_hbm, pk_hbm, pe_hbm, acc_hbm = (next(it) for _ in range(4))
        acc_s = next(it)
        (pe_v, pk_v, idx_v, ridx_v, ewc_v, wu_v, wd_v,
         xr_v, msg_v) = (next(it) for _ in range(9))
        rest = list(it)
        sin = rest[0:NSL]
        sg = rest[NSL:2 * NSL]
        ss = rest[2 * NSL:3 * NSL]

        core = lax.axis_index("c")
        tid = lax.axis_index("s")
        zero16 = jnp.zeros((16,), jnp.float32)
        l16 = lax.iota(jnp.int32, 16)
        ebase = tid * EPT
        qoff = 2 * q + core

        def loads_copy(ci, s):
            return pltpu.make_async_copy(
                pk_hbm.at[:, pl.ds(ebase + ci * K, K)], pk_v.at[s], sin[s])

        def gather_copy(s):
            return pltpu.make_async_copy(
                xt_hbm.at[idx_v.at[s]], xr_v.at[s], sg[s])

        def scat_copy(s):
            return pltpu.make_async_copy(
                msg_v.at[s], acc_s.at[ridx_v.at[s]], ss[s])

        def prep(s):
            for j in range(K // 16):
                sl = pl.ds(16 * j, 16)
                cv = pk_v[s, 1, sl]
                rv = pk_v[s, 0, sl]
                idx_v[s, sl] = cv * 4 + qoff
                ewc_v[s, sl] = plsc.bitcast(pk_v[s, 2, sl], jnp.float32)
                diff = (plsc.load_gather(pe_v, [cv])
                        - plsc.load_gather(pe_v, [rv]))
                wu_v[s, sl] = jnp.maximum(diff, 0.0)
                wd_v[s, sl] = jnp.maximum(-diff, 0.0)

        def copy_ridx(s):
            for j in range(K // 16):
                sl = pl.ds(16 * j, 16)
                ridx_v[s, sl] = pk_v[s, 0, sl]

        def edge_build(s):
            @pl.loop(0, K, unroll=8)
            def _edge(k):
                ks = jnp.full((16,), k, jnp.int32)
                ev = plsc.load_gather(ewc_v.at[s], [ks])
                uv = plsc.load_gather(wu_v.at[s], [ks])
                dv = plsc.load_gather(wd_v.at[s], [ks])
                for c in range(H // 16):
                    xv = xr_v[s, k, pl.ds(16 * c, 16)]
                    msg_v[s, k, pl.ds(16 * c, 16)] = xv * ev
                    msg_v[s, k, pl.ds(H + 16 * c, 16)] = xv * uv
                    msg_v[s, k, pl.ds(2 * H + 16 * c, 16)] = xv * dv
                if with_deg:
                    msg_v[s, k, pl.ds(W3, 16)] = jnp.where(l16 == 0, ev,
                                                           zero16)

        for s in range(NSL):
            loads_copy(s, s).start()

        @pl.loop(0, K)
        def _zb(k):
            for c in range(W // 16):
                msg_v[0, k, pl.ds(16 * c, 16)] = zero16

        r0 = tid * RPT
        nfull = RPT // K
        rem = RPT % K
        for j in range(nfull):
            pltpu.sync_copy(msg_v.at[0], acc_s.at[pl.ds(r0 + K * j, K)])
        if rem:
            pltpu.sync_copy(msg_v.at[0].at[pl.ds(0, rem)],
                            acc_s.at[pl.ds(r0 + RPT - rem, rem)])

        pltpu.sync_copy(pe_hbm, pe_v)
        plsc.subcore_barrier()

        loads_copy(0, 0).wait()
        prep(0)
        gather_copy(0).start()

        @pl.loop(0, T)
        def _step(t):
            for b in range(NSL):
                i = NSL * t + b
                s1 = (b + 1) % NSL

                pl.when(t > 0)(lambda: scat_copy(b).wait())
                copy_ridx(b)

                def prep_next():
                    loads_copy(i + 1, s1).wait()
                    prep(s1)
                    gather_copy(s1).start()

                if b < NSL - 1:
                    prep_next()
                else:
                    pl.when(t < T - 1)(prep_next)

                pl.when(t < T - 1)(lambda: loads_copy(i + 2, b).start())

                gather_copy(b).wait()
                edge_build(b)
                scat_copy(b).start(add=True)

        for b in range(NSL):
            scat_copy(b).wait()
        plsc.subcore_barrier()

        def _wb(src, dst, off):
            for j in range(nfull):
                pltpu.sync_copy(src.at[pl.ds(r0 + K * j, K)],
                                dst.at[pl.ds(off + r0 + K * j, K)])
            if rem:
                pltpu.sync_copy(src.at[pl.ds(r0 + RPT - rem, rem)],
                                dst.at[pl.ds(off + r0 + RPT - rem, rem)])

        _wb(acc_s, acc_hbm, core * N)

    return pl.kernel(
        body,
        out_type=out_type,
        mesh=_MESH,
        compiler_params=pltpu.CompilerParams(use_tc_tiling_on_sc=False,
                                             needs_layout_passes=False),
        scratch_types=scratch,
    )


_hop_q0 = _make_hop(0, True)
_hop_q1 = _make_hop(1, False)


def _fuse_body(*refs):
    (xt_ref, *rest) = refs
    a = rest[:12]
    wx_ref, sp_ref, bf_ref, dl_ref, hb_ref, o_ref = rest[12:]

    x = xt_ref[...]
    dl = dl_ref[...]
    m = jnp.max(dl, axis=0, keepdims=True)
    e = jnp.exp(dl - m)
    dw = e / jnp.sum(e, axis=0, keepdims=True)

    final = jnp.zeros_like(x)
    for p in range(3):
        deg = a[4 * p][...][:, W3:W3 + 1]
        dinv = jnp.where(deg == 0.0, 0.0, 1.0 / deg)
        z = jnp.dot(x, wx_ref[p], preferred_element_type=jnp.float32)
        for u in range(4):
            z = z + jnp.dot(a[4 * p + u][...][:, :W3] * dinv, sp_ref[p, u],
                            preferred_element_type=jnp.float32)
        z = z + bf_ref[p]
        h = jnp.where(z >= 0.0, z, 0.01 * z)
        final = final + h * dw[p] + hb_ref[p]
    o_ref[...] = final


@jax.jit
def _fuse_call(xt, a_in, wx, sp, bf, dl, hb):
    B = N // 10
    row_spec = lambda w: pl.BlockSpec((B, w), lambda i: (i, 0))
    full = lambda s: pl.BlockSpec(s, lambda i: tuple(0 for _ in s))
    a_specs = [row_spec(a.shape[1]) for a in a_in]
    return pl.pallas_call(
        _fuse_body,
        grid=(10,),
        in_specs=(
            [row_spec(C)]
            + a_specs
            + [full((3, C, C)), full((3, 4, W3, C)), full((3, 1, C)),
               full((3, C)), full((3, 1, C))]
        ),
        out_specs=row_spec(C),
        out_shape=jax.ShapeDtypeStruct((N, C), jnp.float32),
    )(xt, *a_in, wx, sp, bf, dl, hb)


@jax.jit
def kernel(x, ei0, ei1, ei2, ew0, ew1, ew2, pe,
           W_in, b_in, d, hop_bias, Wf0, bf0, Wf1, bf1, Wf2, bf2):
    xt = _xt_call(x, W_in.T, b_in.reshape(1, C))
    xt4 = xt.reshape(4 * N, H)
    pe0 = pe[:, 0]

    a_in = []
    for ei, ew in ((ei0, ew0), (ei1, ew1), (ei2, ew2)):
        pk = jnp.concatenate(
            [ei, lax.bitcast_convert_type(ew, jnp.int32)[None]], axis=0)
        acc0 = _hop_q0(xt4, pk, pe0)
        acc1 = _hop_q1(xt4, pk, pe0)
        a_in += [acc0[:N], acc0[N:], acc1[:N], acc1[N:]]

    wx, sp, bf = [], [], []
    for Wf, bfp in ((Wf0, bf0), (Wf1, bf1), (Wf2, bf2)):
        wx.append(Wf[:, :C].T)
        Wa = Wf[:, C:2 * C].T
        Wu = Wf[:, 2 * C::2].T
        Wd = Wf[:, 2 * C + 1::2].T
        sp.append(jnp.stack([
            jnp.concatenate([Wa[H * u:H * (u + 1)],
                             Wu[H * u:H * (u + 1)],
                             Wd[H * u:H * (u + 1)]], axis=0)
            for u in range(4)
        ]))
        bf.append(bfp.reshape(1, C))
    wx = jnp.stack(wx)
    sp = jnp.stack(sp)
    bf = jnp.stack(bf)
    hb = hop_bias.reshape(3, 1, C)

    return _fuse_call(xt, a_in, wx, sp, bf, d, hb)


# speedup vs baseline: 23.1213x; 1.6861x over previous
import functools

import jax
import jax.numpy as jnp
from jax import lax
from jax.experimental import pallas as pl
from jax.experimental.pallas import tpu as pltpu
from jax.experimental.pallas import tpu_sc as plsc

N = 10000
E = 320000
C = 128
H = 32
W3 = 3 * H
K = 80
NT = 16
EPT = E // NT
NCH = EPT // K
NSL = 2
T = NCH // NSL
RPT = N // NT


def _xt_body(x_ref, w_ref, b_ref, o_ref):
    o_ref[...] = (
        jnp.dot(x_ref[...], w_ref[...], preferred_element_type=jnp.float32)
        + b_ref[...]
    )


@jax.jit
def _xt_call(x, wt, b):
    return pl.pallas_call(
        _xt_body,
        grid=(10,),
        in_specs=[
            pl.BlockSpec((N // 10, C), lambda i: (i, 0)),
            pl.BlockSpec((C, C), lambda i: (0, 0)),
            pl.BlockSpec((1, C), lambda i: (0, 0)),
        ],
        out_specs=pl.BlockSpec((N // 10, C), lambda i: (i, 0)),
        out_shape=jax.ShapeDtypeStruct((N, C), jnp.float32),
    )(x, wt, b)


_MESH = plsc.VectorSubcoreMesh(core_axis_name="c", subcore_axis_name="s")


def _make_hop(q, with_deg):
    W = W3 + 16 if with_deg else W3
    out_type = jax.ShapeDtypeStruct((2 * N, W), jnp.float32)
    scratch = [
        pltpu.VMEM_SHARED((N, W), jnp.float32),
        pltpu.VMEM((N,), jnp.float32),
        pltpu.VMEM((NSL, 3, K), jnp.int32),
        pltpu.VMEM((NSL, K), jnp.int32),
        pltpu.VMEM((NSL, K), jnp.int32),
        pltpu.VMEM((NSL, K), jnp.float32),
        pltpu.VMEM((NSL, K), jnp.float32),
        pltpu.VMEM((NSL, K), jnp.float32),
        pltpu.VMEM((NSL, K, H), jnp.float32),
        pltpu.VMEM((NSL, K, W), jnp.float32),
    ] + [pltpu.SemaphoreType.DMA] * (3 * NSL)

    def body(*refs):
        it = iter(refs)
        xt_hbm, pk_hbm, pe_hbm, acc_hbm = (next(it) for _ in range(4))
        acc_s = next(it)
        (pe_v, pk_v, idx_v, ridx_v, ewc_v, wu_v, wd_v,
         xr_v, msg_v) = (next(it) for _ in range(9))
        rest = list(it)
        sin = rest[0:NSL]
        sg = rest[NSL:2 * NSL]
        ss = rest[2 * NSL:3 * NSL]

        core = lax.axis_index("c")
        tid = lax.axis_index("s")
        zero16 = jnp.zeros((16,), jnp.float32)
        l16 = lax.iota(jnp.int32, 16)
        ebase = tid * EPT
        qoff = 2 * q + core

        def loads_copy(ci, s):
            return pltpu.make_async_copy(
                pk_hbm.at[:, pl.ds(ebase + ci * K, K)], pk_v.at[s], sin[s])

        def gather_copy(s):
            return pltpu.make_async_copy(
                xt_hbm.at[idx_v.at[s]], xr_v.at[s], sg[s])

        def scat_copy(s):
            return pltpu.make_async_copy(
                msg_v.at[s], acc_s.at[ridx_v.at[s]], ss[s])

        def prep(s):
            for j in range(K // 16):
                sl = pl.ds(16 * j, 16)
                cv = pk_v[s, 1, sl]
                rv = pk_v[s, 0, sl]
                idx_v[s, sl] = cv * 4 + qoff
                ewc_v[s, sl] = plsc.bitcast(pk_v[s, 2, sl], jnp.float32)
                diff = (plsc.load_gather(pe_v, [cv])
                        - plsc.load_gather(pe_v, [rv]))
                wu_v[s, sl] = jnp.maximum(diff, 0.0)
                wd_v[s, sl] = jnp.maximum(-diff, 0.0)

        def copy_ridx(s):
            for j in range(K // 16):
                sl = pl.ds(16 * j, 16)
                ridx_v[s, sl] = pk_v[s, 0, sl]

        def edge_build(s):
            @plsc.parallel_loop(0, K, 1, unroll=8)
            def _edge(k):
                ks = jnp.full((16,), k, jnp.int32)
                ev = plsc.load_gather(ewc_v.at[s], [ks])
                uv = plsc.load_gather(wu_v.at[s], [ks])
                dv = plsc.load_gather(wd_v.at[s], [ks])
                for c in range(H // 16):
                    xv = xr_v[s, k, pl.ds(16 * c, 16)]
                    msg_v[s, k, pl.ds(16 * c, 16)] = xv * ev
                    msg_v[s, k, pl.ds(H + 16 * c, 16)] = xv * uv
                    msg_v[s, k, pl.ds(2 * H + 16 * c, 16)] = xv * dv
                if with_deg:
                    msg_v[s, k, pl.ds(W3, 16)] = jnp.where(l16 == 0, ev,
                                                           zero16)

        for s in range(NSL):
            loads_copy(s, s).start()

        @pl.loop(0, K)
        def _zb(k):
            for c in range(W // 16):
                msg_v[0, k, pl.ds(16 * c, 16)] = zero16

        r0 = tid * RPT
        nfull = RPT // K
        rem = RPT % K
        for j in range(nfull):
            pltpu.sync_copy(msg_v.at[0], acc_s.at[pl.ds(r0 + K * j, K)])
        if rem:
            pltpu.sync_copy(msg_v.at[0].at[pl.ds(0, rem)],
                            acc_s.at[pl.ds(r0 + RPT - rem, rem)])

        pltpu.sync_copy(pe_hbm, pe_v)
        plsc.subcore_barrier()

        loads_copy(0, 0).wait()
        prep(0)
        gather_copy(0).start()

        @pl.loop(0, T)
        def _step(t):
            for b in range(NSL):
                i = NSL * t + b
                s1 = (b + 1) % NSL

                pl.when(t > 0)(lambda: scat_copy(b).wait())
                copy_ridx(b)

                def prep_next():
                    loads_copy(i + 1, s1).wait()
                    prep(s1)
                    gather_copy(s1).start()

                if b < NSL - 1:
                    prep_next()
                else:
                    pl.when(t < T - 1)(prep_next)

                pl.when(t < T - 1)(lambda: loads_copy(i + 2, b).start())

                gather_copy(b).wait()
                edge_build(b)
                scat_copy(b).start(add=True)

        for b in range(NSL):
            scat_copy(b).wait()
        plsc.subcore_barrier()

        def _wb(src, dst, off):
            for j in range(nfull):
                pltpu.sync_copy(src.at[pl.ds(r0 + K * j, K)],
                                dst.at[pl.ds(off + r0 + K * j, K)])
            if rem:
                pltpu.sync_copy(src.at[pl.ds(r0 + RPT - rem, rem)],
                                dst.at[pl.ds(off + r0 + RPT - rem, rem)])

        _wb(acc_s, acc_hbm, core * N)

    return pl.kernel(
        body,
        out_type=out_type,
        mesh=_MESH,
        compiler_params=pltpu.CompilerParams(use_tc_tiling_on_sc=False,
                                             needs_layout_passes=False),
        scratch_types=scratch,
    )


_hop_q0 = _make_hop(0, True)
_hop_q1 = _make_hop(1, False)


def _fuse_body(*refs):
    (xt_ref, *rest) = refs
    a = rest[:12]
    wx_ref, sp_ref, bf_ref, dl_ref, hb_ref, o_ref = rest[12:]

    x = xt_ref[...]
    dl = dl_ref[...]
    m = jnp.max(dl, axis=0, keepdims=True)
    e = jnp.exp(dl - m)
    dw = e / jnp.sum(e, axis=0, keepdims=True)

    final = jnp.zeros_like(x)
    for p in range(3):
        deg = a[4 * p][...][:, W3:W3 + 1]
        dinv = jnp.where(deg == 0.0, 0.0, 1.0 / deg)
        z = jnp.dot(x, wx_ref[p], preferred_element_type=jnp.float32)
        for u in range(4):
            z = z + jnp.dot(a[4 * p + u][...][:, :W3] * dinv, sp_ref[p, u],
                            preferred_element_type=jnp.float32)
        z = z + bf_ref[p]
        h = jnp.where(z >= 0.0, z, 0.01 * z)
        final = final + h * dw[p] + hb_ref[p]
    o_ref[...] = final


@jax.jit
def _fuse_call(xt, a_in, wx, sp, bf, dl, hb):
    B = N // 10
    row_spec = lambda w: pl.BlockSpec((B, w), lambda i: (i, 0))
    full = lambda s: pl.BlockSpec(s, lambda i: tuple(0 for _ in s))
    a_specs = [row_spec(a.shape[1]) for a in a_in]
    return pl.pallas_call(
        _fuse_body,
        grid=(10,),
        in_specs=(
            [row_spec(C)]
            + a_specs
            + [full((3, C, C)), full((3, 4, W3, C)), full((3, 1, C)),
               full((3, C)), full((3, 1, C))]
        ),
        out_specs=row_spec(C),
        out_shape=jax.ShapeDtypeStruct((N, C), jnp.float32),
    )(xt, *a_in, wx, sp, bf, dl, hb)


@jax.jit
def kernel(x, ei0, ei1, ei2, ew0, ew1, ew2, pe,
           W_in, b_in, d, hop_bias, Wf0, bf0, Wf1, bf1, Wf2, bf2):
    xt = _xt_call(x, W_in.T, b_in.reshape(1, C))
    xt4 = xt.reshape(4 * N, H)
    pe0 = pe[:, 0]

    a_in = []
    for ei, ew in ((ei0, ew0), (ei1, ew1), (ei2, ew2)):
        pk = jnp.concatenate(
            [ei, lax.bitcast_convert_type(ew, jnp.int32)[None]], axis=0)
        acc0 = _hop_q0(xt4, pk, pe0)
        acc1 = _hop_q1(xt4, pk, pe0)
        a_in += [acc0[:N], acc0[N:], acc1[:N], acc1[N:]]

    wx, sp, bf = [], [], []
    for Wf, bfp in ((Wf0, bf0), (Wf1, bf1), (Wf2, bf2)):
        wx.append(Wf[:, :C].T)
        Wa = Wf[:, C:2 * C].T
        Wu = Wf[:, 2 * C::2].T
        Wd = Wf[:, 2 * C + 1::2].T
        sp.append(jnp.stack([
            jnp.concatenate([Wa[H * u:H * (u + 1)],
                             Wu[H * u:H * (u + 1)],
                             Wd[H * u:H * (u + 1)]], axis=0)
            for u in range(4)
        ]))
        bf.append(bfp.reshape(1, C))
    wx = jnp.stack(wx)
    sp = jnp.stack(sp)
    bf = jnp.stack(bf)
    hb = hop_bias.reshape(3, 1, C)

    return _fuse_call(xt, a_in, wx, sp, bf, d, hb)
